# baseline (speedup 1.0000x reference)
import functools

import jax
import jax.numpy as jnp
from jax.experimental import pallas as pl
from jax.experimental.pallas import tpu as pltpu

_CAPACITY = 2.0
_TILE = 128
_NTP = 128


def _gelu_tanh(v):
    return 0.5 * v * (1.0 + jnp.tanh(jnp.sqrt(2.0 / jnp.pi) * (v + 0.044715 * v ** 3)))


def _router_kernel(x_ref, gw_ref, bias_ref, comb_ref, cum_ref, te_ref, rb_ref,
                   stats_ref, *, k):
    x = x_ref[...]
    gw = gw_ref[...]
    bias = bias_ref[...]
    S = x.shape[0]
    E = gw.shape[0]
    logits = jax.lax.dot_general(x, gw, (((1,), (1,)), ((), ())),
                                 preferred_element_type=jnp.float32)
    scores = jax.nn.sigmoid(logits + bias[:, 0][None, :])
    si = jax.lax.bitcast_convert_type(scores, jnp.int32)

    def _tstep(i, t):
        cand = t | (jnp.int32(1) << (30 - i))
        cnt = jnp.sum((si >= cand).astype(jnp.int32), dtype=jnp.int32)
        return jnp.where(cnt >= k, cand, t)

    t = jax.lax.fori_loop(0, 31, _tstep, jnp.int32(0))

    gt = si > t
    eq = si == t
    cg = jnp.sum(gt.astype(jnp.int32), dtype=jnp.int32)
    need = k - cg

    s_iota = jax.lax.broadcasted_iota(jnp.int32, (S, E), 0)
    e_iota = jax.lax.broadcasted_iota(jnp.int32, (S, E), 1)
    fidx = e_iota * S + s_iota

    def _mstep(_, lohi):
        lo, hi = lohi
        mid = (lo + hi) // 2
        cnt = jnp.sum((eq & (fidx <= mid)).astype(jnp.int32), dtype=jnp.int32)
        return jnp.where(cnt >= need, lo, mid + 1), jnp.where(cnt >= need, mid, hi)

    lo, _ = jax.lax.fori_loop(0, 14, _mstep, (jnp.int32(0), jnp.int32(E * S - 1)))

    sel = gt | (eq & (fidx <= lo))
    selb = sel.astype(jnp.bfloat16)
    comb_ref[...] = jnp.where(sel, scores, 0.0)

    ti = jax.lax.broadcasted_iota(jnp.int32, (S, S), 0)
    tj = jax.lax.broadcasted_iota(jnp.int32, (S, S), 1)
    tri = (tj <= ti).astype(jnp.bfloat16)
    cum_ref[...] = jax.lax.dot_general(tri, selb, (((1,), (0,)), ((), ())),
                                       preferred_element_type=jnp.float32)

    ones_col = jnp.ones((S, 1), jnp.float32)
    counts_col = jax.lax.dot_general(sel.astype(jnp.float32), ones_col,
                                     (((0,), (0,)), ((), ())),
                                     preferred_element_type=jnp.float32)
    stats_ref[...] = (counts_col / float(k)) * jnp.ones((E, _NTP), jnp.float32)

    ones_row = jnp.ones((1, S), jnp.float32)
    counts_row = jax.lax.dot_general(ones_row, sel.astype(jnp.float32),
                                     (((1,), (0,)), ((), ())),
                                     preferred_element_type=jnp.float32)
    nt = jnp.floor((counts_row + float(_TILE - 1)) / float(_TILE))
    ui = jax.lax.broadcasted_iota(jnp.int32, (E, E), 0)
    uj = jax.lax.broadcasted_iota(jnp.int32, (E, E), 1)
    u8 = (ui < uj).astype(jnp.float32)
    st = jax.lax.dot_general(nt, u8, (((1,), (0,)), ((), ())),
                             preferred_element_type=jnp.float32)
    en = st + nt
    tl = jax.lax.broadcasted_iota(jnp.int32, (_NTP, E), 0).astype(jnp.float32)
    active = ((tl >= st) & (tl < en)).astype(jnp.float32)
    e_lane = jax.lax.broadcasted_iota(jnp.int32, (_NTP, E), 1).astype(jnp.float32)
    te = jnp.sum(active * e_lane, axis=1, keepdims=True)
    rb = jnp.sum(active * (tl - st), axis=1, keepdims=True) * float(_TILE)
    vld = jnp.sum(active, axis=1, keepdims=True)
    te_ref[...] = jnp.where(vld > 0, te, float(E - 1)).astype(jnp.int32)
    rb_ref[...] = jnp.where(vld > 0, rb, -1.0).astype(jnp.int32)


def _router_call(x_flat, gate_weight, expert_bias, k):
    S, D = x_flat.shape
    E = gate_weight.shape[0]
    return pl.pallas_call(
        functools.partial(_router_kernel, k=k),
        out_shape=(
            jax.ShapeDtypeStruct((S, E), jnp.float32),
            jax.ShapeDtypeStruct((S, E), jnp.float32),
            jax.ShapeDtypeStruct((_NTP, 1), jnp.int32),
            jax.ShapeDtypeStruct((_NTP, 1), jnp.int32),
            jax.ShapeDtypeStruct((E, _NTP), jnp.float32),
        ),
    )(x_flat, gate_weight, expert_bias)


def _experts_kernel(te_ref, rb_ref, x_ref, w1_ref, b1_ref, w2_ref, b2_ref,
                    comb_ref, cum_ref, out_ref):
    t = pl.program_id(0)

    @pl.when(t == 0)
    def _():
        out_ref[...] = jnp.zeros_like(out_ref)

    @pl.when(rb_ref[t] >= 0)
    def _():
        rbv = rb_ref[t].astype(jnp.float32)
        cum = cum_ref[0]
        comb = comb_ref[0]
        cols = jax.lax.broadcasted_iota(jnp.int32, (1, _TILE), 1).astype(jnp.float32)
        pt = ((cum == (rbv + 1.0 + cols)) & (comb > 0.0))
        ptb = pt.astype(jnp.bfloat16)

        xg = jax.lax.dot_general(ptb, x_ref[...], (((0,), (0,)), ((), ())),
                                 preferred_element_type=jnp.float32)
        h = jax.lax.dot_general(xg.astype(jnp.bfloat16), w1_ref[0],
                                (((1,), (1,)), ((), ())),
                                preferred_element_type=jnp.float32)
        h = _gelu_tanh(h + b1_ref[0])
        y = jax.lax.dot_general(h.astype(jnp.bfloat16), w2_ref[0],
                                (((1,), (1,)), ((), ())),
                                preferred_element_type=jnp.float32)
        g = jax.lax.dot_general(pt.astype(jnp.float32), comb,
                                (((0,), (0,)), ((), ())),
                                preferred_element_type=jnp.float32,
                                precision=jax.lax.Precision.HIGHEST)
        yg = (g * (y + b2_ref[0])).astype(jnp.bfloat16)
        out_ref[...] += jax.lax.dot_general(
            ptb, yg, (((1,), (0,)), ((), ())),
            preferred_element_type=jnp.float32)


def _experts_call(x_flat, W1, b1, W2, b2, comb, cum, te, rb, nt_grid):
    S, D = x_flat.shape
    E, DFF, _ = W1.shape
    grid_spec = pltpu.PrefetchScalarGridSpec(
        num_scalar_prefetch=2,
        grid=(nt_grid,),
        in_specs=[
            pl.BlockSpec((S, D), lambda t, te, rb: (0, 0)),
            pl.BlockSpec((1, DFF, D), lambda t, te, rb: (te[t], 0, 0)),
            pl.BlockSpec((1, 1, DFF), lambda t, te, rb: (te[t], 0, 0)),
            pl.BlockSpec((1, D, DFF), lambda t, te, rb: (te[t], 0, 0)),
            pl.BlockSpec((1, 1, D), lambda t, te, rb: (te[t], 0, 0)),
            pl.BlockSpec((1, S, 1), lambda t, te, rb: (te[t], 0, 0)),
            pl.BlockSpec((1, S, 1), lambda t, te, rb: (te[t], 0, 0)),
        ],
        out_specs=pl.BlockSpec((S, D), lambda t, te, rb: (0, 0)),
    )
    return pl.pallas_call(
        _experts_kernel,
        grid_spec=grid_spec,
        out_shape=jax.ShapeDtypeStruct((S, D), jnp.float32),
        compiler_params=pltpu.CompilerParams(
            vmem_limit_bytes=100 * 1024 * 1024),
    )(te, rb, x_flat.astype(jnp.bfloat16),
      W1.astype(jnp.bfloat16), b1.reshape(E, 1, DFF),
      W2.astype(jnp.bfloat16), b2.reshape(E, 1, D),
      comb.T.reshape(E, S, 1), cum.T.reshape(E, S, 1))


def kernel(x, gate_weight, expert_bias, W1, b1, W2, b2):
    Bsz, seq, D = x.shape
    E = gate_weight.shape[0]
    x_flat = x.reshape(-1, D)
    S = x_flat.shape[0]
    k = int(S * _CAPACITY)
    nt_grid = k // _TILE + E

    comb, cum, te2, rb2, stats = _router_call(x_flat, gate_weight, expert_bias, k)
    te = te2[:nt_grid, 0]
    rb = rb2[:nt_grid, 0]
    out = _experts_call(x_flat, W1, b1, W2, b2, comb, cum, te, rb, nt_grid)

    x_out = out.reshape(Bsz, seq, D)
    token_each_expert = stats[:, 0]
    ones_like_mean = jnp.ones((E,), dtype=x.dtype)
    return (x_out, token_each_expert, ones_like_mean)


# speedup vs baseline: 1.7092x; 1.7092x over previous
import functools

import jax
import jax.numpy as jnp
from jax.experimental import pallas as pl
from jax.experimental.pallas import tpu as pltpu

_CAPACITY = 2.0
_TILE = 256
_NTP = 128


def _gelu_tanh(v):
    return 0.5 * v * (1.0 + jnp.tanh(jnp.sqrt(2.0 / jnp.pi) * (v + 0.044715 * v ** 3)))


def _router_kernel(x_ref, gw_ref, bias_ref, comb_ref, cum_ref, te_ref, rb_ref,
                   stats_ref, *, k):
    x = x_ref[...]
    gw = gw_ref[...]
    bias = bias_ref[...]
    S = x.shape[0]
    E = gw.shape[0]
    logits = jax.lax.dot_general(x, gw, (((1,), (1,)), ((), ())),
                                 preferred_element_type=jnp.float32)
    scores = jax.nn.sigmoid(logits + bias[:, 0][None, :])
    si = jax.lax.bitcast_convert_type(scores, jnp.int32)

    def _tstep(i, t):
        cand = t | (jnp.int32(1) << (30 - i))
        cnt = jnp.sum((si >= cand).astype(jnp.int32), dtype=jnp.int32)
        return jnp.where(cnt >= k, cand, t)

    t = jax.lax.fori_loop(0, 31, _tstep, jnp.int32(0))

    gt = si > t
    eq = si == t
    cg = jnp.sum(gt.astype(jnp.int32), dtype=jnp.int32)
    need = k - cg

    s_iota = jax.lax.broadcasted_iota(jnp.int32, (S, E), 0)
    e_iota = jax.lax.broadcasted_iota(jnp.int32, (S, E), 1)
    fidx = e_iota * S + s_iota

    def _mstep(_, lohi):
        lo, hi = lohi
        mid = (lo + hi) // 2
        cnt = jnp.sum((eq & (fidx <= mid)).astype(jnp.int32), dtype=jnp.int32)
        return jnp.where(cnt >= need, lo, mid + 1), jnp.where(cnt >= need, mid, hi)

    lo, _ = jax.lax.fori_loop(0, 14, _mstep, (jnp.int32(0), jnp.int32(E * S - 1)))

    sel = gt | (eq & (fidx <= lo))
    selb = sel.astype(jnp.bfloat16)
    comb_ref[...] = jnp.where(sel, scores, 0.0)

    ti = jax.lax.broadcasted_iota(jnp.int32, (S, S), 0)
    tj = jax.lax.broadcasted_iota(jnp.int32, (S, S), 1)
    tri = (tj <= ti).astype(jnp.bfloat16)
    cum_ref[...] = jax.lax.dot_general(tri, selb, (((1,), (0,)), ((), ())),
                                       preferred_element_type=jnp.float32)

    ones_col = jnp.ones((S, 1), jnp.float32)
    counts_col = jax.lax.dot_general(sel.astype(jnp.float32), ones_col,
                                     (((0,), (0,)), ((), ())),
                                     preferred_element_type=jnp.float32)
    stats_ref[...] = (counts_col / float(k)) * jnp.ones((E, _NTP), jnp.float32)

    ones_row = jnp.ones((1, S), jnp.float32)
    counts_row = jax.lax.dot_general(ones_row, sel.astype(jnp.float32),
                                     (((1,), (0,)), ((), ())),
                                     preferred_element_type=jnp.float32)
    nt = jnp.floor((counts_row + float(_TILE - 1)) / float(_TILE))
    ui = jax.lax.broadcasted_iota(jnp.int32, (E, E), 0)
    uj = jax.lax.broadcasted_iota(jnp.int32, (E, E), 1)
    u8 = (ui < uj).astype(jnp.float32)
    st = jax.lax.dot_general(nt, u8, (((1,), (0,)), ((), ())),
                             preferred_element_type=jnp.float32)
    en = st + nt
    tl = jax.lax.broadcasted_iota(jnp.int32, (_NTP, E), 0).astype(jnp.float32)
    active = ((tl >= st) & (tl < en)).astype(jnp.float32)
    e_lane = jax.lax.broadcasted_iota(jnp.int32, (_NTP, E), 1).astype(jnp.float32)
    te = jnp.sum(active * e_lane, axis=1, keepdims=True)
    rb = jnp.sum(active * (tl - st), axis=1, keepdims=True) * float(_TILE)
    vld = jnp.sum(active, axis=1, keepdims=True)
    te_ref[...] = jnp.where(vld > 0, te, float(E - 1)).astype(jnp.int32)
    rb_ref[...] = jnp.where(vld > 0, rb, -1.0).astype(jnp.int32)


def _router_call(x_flat, gate_weight, expert_bias, k):
    S, D = x_flat.shape
    E = gate_weight.shape[0]
    return pl.pallas_call(
        functools.partial(_router_kernel, k=k),
        out_shape=(
            jax.ShapeDtypeStruct((S, E), jnp.float32),
            jax.ShapeDtypeStruct((S, E), jnp.float32),
            jax.ShapeDtypeStruct((_NTP, 1), jnp.int32),
            jax.ShapeDtypeStruct((_NTP, 1), jnp.int32),
            jax.ShapeDtypeStruct((E, _NTP), jnp.float32),
        ),
    )(x_flat, gate_weight, expert_bias)


def _experts_kernel(te_ref, rb_ref, x_ref, w1_ref, b1_ref, w2_ref, b2_ref,
                    comb_ref, cum_ref, out_ref, w1b_ref, w2b_ref):
    t = pl.program_id(0)

    @pl.when(t == 0)
    def _():
        out_ref[...] = jnp.zeros_like(out_ref)

    prev_te = jnp.where(t == 0, -1, te_ref[jnp.maximum(t - 1, 0)])

    @pl.when((rb_ref[t] >= 0) & (te_ref[t] != prev_te))
    def _():
        w1b_ref[...] = w1_ref[0].astype(jnp.bfloat16)
        w2b_ref[...] = w2_ref[0].astype(jnp.bfloat16)

    @pl.when(rb_ref[t] >= 0)
    def _():
        rbv = rb_ref[t].astype(jnp.float32)
        cum = cum_ref[0]
        comb = comb_ref[0]
        cols = jax.lax.broadcasted_iota(jnp.int32, (1, _TILE), 1).astype(jnp.float32)
        pt = ((cum == (rbv + 1.0 + cols)) & (comb > 0.0))
        ptb = pt.astype(jnp.bfloat16)
        ptg = jnp.where(pt, comb, 0.0).astype(jnp.bfloat16)

        xg = jax.lax.dot_general(ptb, x_ref[...], (((0,), (0,)), ((), ())),
                                 preferred_element_type=jnp.float32)
        h = jax.lax.dot_general(xg.astype(jnp.bfloat16), w1b_ref[...],
                                (((1,), (1,)), ((), ())),
                                preferred_element_type=jnp.float32)
        h = _gelu_tanh(h + b1_ref[0])
        y = jax.lax.dot_general(h.astype(jnp.bfloat16), w2b_ref[...],
                                (((1,), (1,)), ((), ())),
                                preferred_element_type=jnp.float32)
        yb = (y + b2_ref[0]).astype(jnp.bfloat16)
        out_ref[...] += jax.lax.dot_general(
            ptg, yb, (((1,), (0,)), ((), ())),
            preferred_element_type=jnp.float32)


def _experts_call(x_flat, W1, b1, W2, b2, comb, cum, te, rb, nt_grid):
    S, D = x_flat.shape
    E, DFF, _ = W1.shape
    grid_spec = pltpu.PrefetchScalarGridSpec(
        num_scalar_prefetch=2,
        grid=(nt_grid,),
        in_specs=[
            pl.BlockSpec((S, D), lambda t, te, rb: (0, 0)),
            pl.BlockSpec((1, DFF, D), lambda t, te, rb: (te[t], 0, 0)),
            pl.BlockSpec((1, 1, DFF), lambda t, te, rb: (te[t], 0, 0)),
            pl.BlockSpec((1, D, DFF), lambda t, te, rb: (te[t], 0, 0)),
            pl.BlockSpec((1, 1, D), lambda t, te, rb: (te[t], 0, 0)),
            pl.BlockSpec((1, S, 1), lambda t, te, rb: (te[t], 0, 0)),
            pl.BlockSpec((1, S, 1), lambda t, te, rb: (te[t], 0, 0)),
        ],
        out_specs=pl.BlockSpec((S, D), lambda t, te, rb: (0, 0)),
        scratch_shapes=[
            pltpu.VMEM((DFF, D), jnp.bfloat16),
            pltpu.VMEM((D, DFF), jnp.bfloat16),
        ],
    )
    return pl.pallas_call(
        _experts_kernel,
        grid_spec=grid_spec,
        out_shape=jax.ShapeDtypeStruct((S, D), jnp.float32),
        compiler_params=pltpu.CompilerParams(
            vmem_limit_bytes=100 * 1024 * 1024),
    )(te, rb, x_flat.astype(jnp.bfloat16),
      W1, b1.reshape(E, 1, DFF),
      W2, b2.reshape(E, 1, D),
      comb.T.reshape(E, S, 1), cum.T.reshape(E, S, 1))


def kernel(x, gate_weight, expert_bias, W1, b1, W2, b2):
    Bsz, seq, D = x.shape
    E = gate_weight.shape[0]
    x_flat = x.reshape(-1, D)
    S = x_flat.shape[0]
    k = int(S * _CAPACITY)
    nt_grid = k // _TILE + E

    comb, cum, te2, rb2, stats = _router_call(x_flat, gate_weight, expert_bias, k)
    te = te2[:nt_grid, 0]
    rb = rb2[:nt_grid, 0]
    out = _experts_call(x_flat, W1, b1, W2, b2, comb, cum, te, rb, nt_grid)

    x_out = out.reshape(Bsz, seq, D)
    token_each_expert = stats[:, 0]
    ones_like_mean = jnp.ones((E,), dtype=x.dtype)
    return (x_out, token_each_expert, ones_like_mean)


# speedup vs baseline: 2.0895x; 1.2225x over previous
import functools

import jax
import jax.numpy as jnp
from jax.experimental import pallas as pl
from jax.experimental.pallas import tpu as pltpu

_CAPACITY = 2.0
_TILE = 256
_NTP = 128


def _gelu_tanh(v):
    return 0.5 * v * (1.0 + jnp.tanh(jnp.sqrt(2.0 / jnp.pi) * (v + 0.044715 * v ** 3)))


def _router_kernel(x_ref, gw_ref, bias_ref, comb_ref, cum_ref, te_ref, rb_ref,
                   stats_ref, *, k):
    x = x_ref[...]
    gw = gw_ref[...]
    bias = bias_ref[...]
    S = x.shape[0]
    E = gw.shape[0]
    logits = jax.lax.dot_general(x, gw, (((1,), (1,)), ((), ())),
                                 preferred_element_type=jnp.float32)
    scores = jax.nn.sigmoid(logits + bias[:, 0][None, :])
    si = jax.lax.bitcast_convert_type(scores, jnp.int32)

    def _tstep(i, t):
        cand = t | (jnp.int32(1) << (30 - i))
        cnt = jnp.sum((si >= cand).astype(jnp.int32), dtype=jnp.int32)
        return jnp.where(cnt >= k, cand, t)

    t = jax.lax.fori_loop(0, 31, _tstep, jnp.int32(0))

    gt = si > t
    eq = si == t
    cg = jnp.sum(gt.astype(jnp.int32), dtype=jnp.int32)
    need = k - cg

    s_iota = jax.lax.broadcasted_iota(jnp.int32, (S, E), 0)
    e_iota = jax.lax.broadcasted_iota(jnp.int32, (S, E), 1)
    fidx = e_iota * S + s_iota

    def _mstep(_, lohi):
        lo, hi = lohi
        mid = (lo + hi) // 2
        cnt = jnp.sum((eq & (fidx <= mid)).astype(jnp.int32), dtype=jnp.int32)
        return jnp.where(cnt >= need, lo, mid + 1), jnp.where(cnt >= need, mid, hi)

    lo, _ = jax.lax.fori_loop(0, 14, _mstep, (jnp.int32(0), jnp.int32(E * S - 1)))

    sel = gt | (eq & (fidx <= lo))
    selb = sel.astype(jnp.bfloat16)
    comb_ref[...] = jnp.where(sel, scores, 0.0)

    ti = jax.lax.broadcasted_iota(jnp.int32, (S, S), 0)
    tj = jax.lax.broadcasted_iota(jnp.int32, (S, S), 1)
    tri = (tj <= ti).astype(jnp.bfloat16)
    cum_ref[...] = jax.lax.dot_general(tri, selb, (((1,), (0,)), ((), ())),
                                       preferred_element_type=jnp.float32)

    ones_col = jnp.ones((S, 1), jnp.float32)
    counts_col = jax.lax.dot_general(sel.astype(jnp.float32), ones_col,
                                     (((0,), (0,)), ((), ())),
                                     preferred_element_type=jnp.float32)
    stats_ref[...] = (counts_col / float(k)) * jnp.ones((E, _NTP), jnp.float32)

    ones_row = jnp.ones((1, S), jnp.float32)
    counts_row = jax.lax.dot_general(ones_row, sel.astype(jnp.float32),
                                     (((1,), (0,)), ((), ())),
                                     preferred_element_type=jnp.float32)
    nt = jnp.floor((counts_row + float(_TILE - 1)) / float(_TILE))
    ui = jax.lax.broadcasted_iota(jnp.int32, (E, E), 0)
    uj = jax.lax.broadcasted_iota(jnp.int32, (E, E), 1)
    u8 = (ui < uj).astype(jnp.float32)
    st = jax.lax.dot_general(nt, u8, (((1,), (0,)), ((), ())),
                             preferred_element_type=jnp.float32)
    en = st + nt
    tl = jax.lax.broadcasted_iota(jnp.int32, (_NTP, E), 0).astype(jnp.float32)
    active = ((tl >= st) & (tl < en)).astype(jnp.float32)
    e_lane = jax.lax.broadcasted_iota(jnp.int32, (_NTP, E), 1).astype(jnp.float32)
    te = jnp.sum(active * e_lane, axis=1, keepdims=True)
    rb = jnp.sum(active * (tl - st), axis=1, keepdims=True) * float(_TILE)
    vld = jnp.sum(active, axis=1, keepdims=True)
    te_ref[...] = jnp.where(vld > 0, te, float(E - 1)).astype(jnp.int32)
    rb_ref[...] = jnp.where(vld > 0, rb, -1.0).astype(jnp.int32)


def _router_call(x_flat, gate_weight, expert_bias, k):
    S, D = x_flat.shape
    E = gate_weight.shape[0]
    return pl.pallas_call(
        functools.partial(_router_kernel, k=k),
        out_shape=(
            jax.ShapeDtypeStruct((S, E), jnp.float32),
            jax.ShapeDtypeStruct((S, E), jnp.float32),
            jax.ShapeDtypeStruct((_NTP, 1), jnp.int32),
            jax.ShapeDtypeStruct((_NTP, 1), jnp.int32),
            jax.ShapeDtypeStruct((E, _NTP), jnp.float32),
        ),
    )(x_flat, gate_weight, expert_bias)


def _experts_kernel(te_ref, rb_ref, x_ref, w1_ref, b1_ref, w2_ref, b2_ref,
                    comb_ref, cum_ref, out_ref, w1b_ref, w2b_ref):
    t = pl.program_id(0)

    @pl.when(t == 0)
    def _():
        out_ref[...] = jnp.zeros_like(out_ref)

    prev_te = jnp.where(t == 0, -1, te_ref[jnp.maximum(t - 1, 0)])

    @pl.when((rb_ref[t] >= 0) & (te_ref[t] != prev_te))
    def _():
        w1b_ref[...] = w1_ref[0].astype(jnp.bfloat16)
        w2b_ref[...] = w2_ref[0].astype(jnp.bfloat16)

    @pl.when(rb_ref[t] >= 0)
    def _():
        rbv = rb_ref[t].astype(jnp.float32)
        cum = cum_ref[0]
        comb = comb_ref[0]
        rows = jax.lax.broadcasted_iota(jnp.int32, (_TILE, 1), 0).astype(jnp.float32)
        p = ((cum == (rbv + 1.0 + rows)) & (comb > 0.0))
        pb = p.astype(jnp.bfloat16)
        pg = jnp.where(p, comb, 0.0).astype(jnp.bfloat16)

        xg = jax.lax.dot_general(pb, x_ref[...], (((1,), (0,)), ((), ())),
                                 preferred_element_type=jnp.float32)
        h = jax.lax.dot_general(xg.astype(jnp.bfloat16), w1b_ref[...],
                                (((1,), (1,)), ((), ())),
                                preferred_element_type=jnp.float32)
        h = _gelu_tanh(h + b1_ref[0])
        y = jax.lax.dot_general(h.astype(jnp.bfloat16), w2b_ref[...],
                                (((1,), (1,)), ((), ())),
                                preferred_element_type=jnp.float32)
        yb = (y + b2_ref[0]).astype(jnp.bfloat16)
        out_ref[...] += jax.lax.dot_general(
            pg, yb, (((0,), (0,)), ((), ())),
            preferred_element_type=jnp.float32)


def _experts_call(x_flat, W1, b1, W2, b2, comb, cum, te, rb, nt_grid):
    S, D = x_flat.shape
    E, DFF, _ = W1.shape
    grid_spec = pltpu.PrefetchScalarGridSpec(
        num_scalar_prefetch=2,
        grid=(nt_grid,),
        in_specs=[
            pl.BlockSpec((S, D), lambda t, te, rb: (0, 0)),
            pl.BlockSpec((1, DFF, D), lambda t, te, rb: (te[t], 0, 0)),
            pl.BlockSpec((1, 1, DFF), lambda t, te, rb: (te[t], 0, 0)),
            pl.BlockSpec((1, D, DFF), lambda t, te, rb: (te[t], 0, 0)),
            pl.BlockSpec((1, 1, D), lambda t, te, rb: (te[t], 0, 0)),
            pl.BlockSpec((1, 1, S), lambda t, te, rb: (te[t], 0, 0)),
            pl.BlockSpec((1, 1, S), lambda t, te, rb: (te[t], 0, 0)),
        ],
        out_specs=pl.BlockSpec((S, D), lambda t, te, rb: (0, 0)),
        scratch_shapes=[
            pltpu.VMEM((DFF, D), jnp.bfloat16),
            pltpu.VMEM((D, DFF), jnp.bfloat16),
        ],
    )
    return pl.pallas_call(
        _experts_kernel,
        grid_spec=grid_spec,
        out_shape=jax.ShapeDtypeStruct((S, D), jnp.float32),
        compiler_params=pltpu.CompilerParams(
            vmem_limit_bytes=100 * 1024 * 1024),
    )(te, rb, x_flat.astype(jnp.bfloat16),
      W1, b1.reshape(E, 1, DFF),
      W2, b2.reshape(E, 1, D),
      comb.T.reshape(E, 1, S), cum.T.reshape(E, 1, S))


def kernel(x, gate_weight, expert_bias, W1, b1, W2, b2):
    Bsz, seq, D = x.shape
    E = gate_weight.shape[0]
    x_flat = x.reshape(-1, D)
    S = x_flat.shape[0]
    k = int(S * _CAPACITY)
    nt_grid = k // _TILE + E

    comb, cum, te2, rb2, stats = _router_call(x_flat, gate_weight, expert_bias, k)
    te = te2[:nt_grid, 0]
    rb = rb2[:nt_grid, 0]
    out = _experts_call(x_flat, W1, b1, W2, b2, comb, cum, te, rb, nt_grid)

    x_out = out.reshape(Bsz, seq, D)
    token_each_expert = stats[:, 0]
    ones_like_mean = jnp.ones((E,), dtype=x.dtype)
    return (x_out, token_each_expert, ones_like_mean)


# speedup vs baseline: 2.0937x; 1.0020x over previous
import functools

import jax
import jax.numpy as jnp
from jax.experimental import pallas as pl
from jax.experimental.pallas import tpu as pltpu

_CAPACITY = 2.0
_TILE = 256
_NTP = 128


def _gelu_tanh(v):
    return 0.5 * v * (1.0 + jnp.tanh(jnp.sqrt(2.0 / jnp.pi) * (v + 0.044715 * v ** 3)))


def _router_kernel(x_ref, gw_ref, bias_ref, comb_ref, cum_ref, te_ref, rb_ref,
                   stats_ref, *, k):
    x = x_ref[...]
    gw = gw_ref[...]
    bias = bias_ref[...]
    S = x.shape[0]
    E = gw.shape[0]
    logits = jax.lax.dot_general(x, gw, (((1,), (1,)), ((), ())),
                                 preferred_element_type=jnp.float32)
    scores = jax.nn.sigmoid(logits + bias[:, 0][None, :])
    si = jax.lax.bitcast_convert_type(scores, jnp.int32)

    def _tstep(i, t):
        cand = t | (jnp.int32(1) << (30 - i))
        cnt = jnp.sum((si >= cand).astype(jnp.int32), dtype=jnp.int32)
        return jnp.where(cnt >= k, cand, t)

    t = jax.lax.fori_loop(0, 31, _tstep, jnp.int32(0))

    gt = si > t
    eq = si == t
    cg = jnp.sum(gt.astype(jnp.int32), dtype=jnp.int32)
    need = k - cg

    s_iota = jax.lax.broadcasted_iota(jnp.int32, (S, E), 0)
    e_iota = jax.lax.broadcasted_iota(jnp.int32, (S, E), 1)
    fidx = e_iota * S + s_iota

    def _mstep(_, lohi):
        lo, hi = lohi
        mid = (lo + hi) // 2
        cnt = jnp.sum((eq & (fidx <= mid)).astype(jnp.int32), dtype=jnp.int32)
        return jnp.where(cnt >= need, lo, mid + 1), jnp.where(cnt >= need, mid, hi)

    lo, _ = jax.lax.fori_loop(0, 14, _mstep, (jnp.int32(0), jnp.int32(E * S - 1)))

    sel = gt | (eq & (fidx <= lo))
    selb = sel.astype(jnp.bfloat16)
    comb_ref[...] = jnp.where(sel, scores, 0.0)

    ti = jax.lax.broadcasted_iota(jnp.int32, (S, S), 0)
    tj = jax.lax.broadcasted_iota(jnp.int32, (S, S), 1)
    tri = (tj <= ti).astype(jnp.bfloat16)
    cum_ref[...] = jax.lax.dot_general(tri, selb, (((1,), (0,)), ((), ())),
                                       preferred_element_type=jnp.float32)

    ones_col = jnp.ones((S, 1), jnp.float32)
    counts_col = jax.lax.dot_general(sel.astype(jnp.float32), ones_col,
                                     (((0,), (0,)), ((), ())),
                                     preferred_element_type=jnp.float32)
    stats_ref[...] = (counts_col / float(k)) * jnp.ones((E, _NTP), jnp.float32)

    ones_row = jnp.ones((1, S), jnp.float32)
    counts_row = jax.lax.dot_general(ones_row, sel.astype(jnp.float32),
                                     (((1,), (0,)), ((), ())),
                                     preferred_element_type=jnp.float32)
    nt = jnp.floor((counts_row + float(_TILE - 1)) / float(_TILE))
    ui = jax.lax.broadcasted_iota(jnp.int32, (E, E), 0)
    uj = jax.lax.broadcasted_iota(jnp.int32, (E, E), 1)
    u8 = (ui < uj).astype(jnp.float32)
    st = jax.lax.dot_general(nt, u8, (((1,), (0,)), ((), ())),
                             preferred_element_type=jnp.float32)
    en = st + nt
    tl = jax.lax.broadcasted_iota(jnp.int32, (_NTP, E), 0).astype(jnp.float32)
    active = ((tl >= st) & (tl < en)).astype(jnp.float32)
    e_lane = jax.lax.broadcasted_iota(jnp.int32, (_NTP, E), 1).astype(jnp.float32)
    te = jnp.sum(active * e_lane, axis=1, keepdims=True)
    rb = jnp.sum(active * (tl - st), axis=1, keepdims=True) * float(_TILE)
    vld = jnp.sum(active, axis=1, keepdims=True)
    te_ref[...] = jnp.where(vld > 0, te, float(E - 1)).astype(jnp.int32)
    rb_ref[...] = jnp.where(vld > 0, rb, -1.0).astype(jnp.int32)


def _router_call(x_flat, gate_weight, expert_bias, k):
    S, D = x_flat.shape
    E = gate_weight.shape[0]
    return pl.pallas_call(
        functools.partial(_router_kernel, k=k),
        out_shape=(
            jax.ShapeDtypeStruct((S, E), jnp.float32),
            jax.ShapeDtypeStruct((S, E), jnp.float32),
            jax.ShapeDtypeStruct((_NTP, 1), jnp.int32),
            jax.ShapeDtypeStruct((_NTP, 1), jnp.int32),
            jax.ShapeDtypeStruct((E, _NTP), jnp.float32),
        ),
    )(x_flat, gate_weight, expert_bias)


def _experts_kernel(te_ref, rb_ref, x_ref, w1_ref, b1_ref, w2_ref, b2_ref,
                    comb_ref, cum_ref, out_ref, w1b_ref, w2b_ref):
    t = pl.program_id(0)

    @pl.when(t == 0)
    def _():
        out_ref[...] = jnp.zeros_like(out_ref)

    prev_te = jnp.where(t == 0, -1, te_ref[jnp.maximum(t - 1, 0)])

    @pl.when((rb_ref[t] >= 0) & (te_ref[t] != prev_te))
    def _():
        w1b_ref[...] = w1_ref[0].astype(jnp.bfloat16)
        w2b_ref[...] = w2_ref[0].astype(jnp.bfloat16)

    @pl.when(rb_ref[t] >= 0)
    def _():
        rbv = rb_ref[t].astype(jnp.float32)
        cum = cum_ref[0]
        comb = comb_ref[0]
        rows = jax.lax.broadcasted_iota(jnp.int32, (_TILE, 1), 0).astype(jnp.float32)
        p = ((cum == (rbv + 1.0 + rows)) & (comb > 0.0))
        pb = p.astype(jnp.bfloat16)
        pg = jnp.where(p, comb, 0.0).astype(jnp.bfloat16)

        xg = jax.lax.dot_general(pb, x_ref[...], (((1,), (0,)), ((), ())),
                                 preferred_element_type=jnp.float32)
        h = jax.lax.dot_general(xg.astype(jnp.bfloat16), w1b_ref[...],
                                (((1,), (1,)), ((), ())),
                                preferred_element_type=jnp.float32)
        h = _gelu_tanh((h + b1_ref[0]).astype(jnp.bfloat16))
        y = jax.lax.dot_general(h, w2b_ref[...],
                                (((1,), (1,)), ((), ())),
                                preferred_element_type=jnp.float32)
        yb = (y + b2_ref[0]).astype(jnp.bfloat16)
        out_ref[...] += jax.lax.dot_general(
            pg, yb, (((0,), (0,)), ((), ())),
            preferred_element_type=jnp.float32)


def _experts_call(x_flat, W1, b1, W2, b2, comb, cum, te, rb, nt_grid):
    S, D = x_flat.shape
    E, DFF, _ = W1.shape
    grid_spec = pltpu.PrefetchScalarGridSpec(
        num_scalar_prefetch=2,
        grid=(nt_grid,),
        in_specs=[
            pl.BlockSpec((S, D), lambda t, te, rb: (0, 0)),
            pl.BlockSpec((1, DFF, D), lambda t, te, rb: (te[t], 0, 0)),
            pl.BlockSpec((1, 1, DFF), lambda t, te, rb: (te[t], 0, 0)),
            pl.BlockSpec((1, D, DFF), lambda t, te, rb: (te[t], 0, 0)),
            pl.BlockSpec((1, 1, D), lambda t, te, rb: (te[t], 0, 0)),
            pl.BlockSpec((1, 1, S), lambda t, te, rb: (te[t], 0, 0)),
            pl.BlockSpec((1, 1, S), lambda t, te, rb: (te[t], 0, 0)),
        ],
        out_specs=pl.BlockSpec((S, D), lambda t, te, rb: (0, 0)),
        scratch_shapes=[
            pltpu.VMEM((DFF, D), jnp.bfloat16),
            pltpu.VMEM((D, DFF), jnp.bfloat16),
        ],
    )
    return pl.pallas_call(
        _experts_kernel,
        grid_spec=grid_spec,
        out_shape=jax.ShapeDtypeStruct((S, D), jnp.float32),
        compiler_params=pltpu.CompilerParams(
            vmem_limit_bytes=100 * 1024 * 1024),
    )(te, rb, x_flat.astype(jnp.bfloat16),
      W1, b1.reshape(E, 1, DFF),
      W2, b2.reshape(E, 1, D),
      comb.T.reshape(E, 1, S), cum.T.reshape(E, 1, S))


def kernel(x, gate_weight, expert_bias, W1, b1, W2, b2):
    Bsz, seq, D = x.shape
    E = gate_weight.shape[0]
    x_flat = x.reshape(-1, D)
    S = x_flat.shape[0]
    k = int(S * _CAPACITY)
    nt_grid = k // _TILE + E

    comb, cum, te2, rb2, stats = _router_call(x_flat, gate_weight, expert_bias, k)
    te = te2[:nt_grid, 0]
    rb = rb2[:nt_grid, 0]
    out = _experts_call(x_flat, W1, b1, W2, b2, comb, cum, te, rb, nt_grid)

    x_out = out.reshape(Bsz, seq, D)
    token_each_expert = stats[:, 0]
    ones_like_mean = jnp.ones((E,), dtype=x.dtype)
    return (x_out, token_each_expert, ones_like_mean)
